# Initial kernel scaffold; baseline (speedup 1.0000x reference)
import jax
import jax.numpy as jnp
from jax.experimental import pallas as pl


def kernel(q, kv_src, seg_id, q_pos_ids, kv_pos_ids, Wq, Wkv, Wo):
    raise NotImplementedError("write your pallas kernel here")


# speedup vs baseline: 16.8668x; 16.8668x over previous
import functools

import jax
import jax.numpy as jnp
from jax.experimental import pallas as pl
from jax.experimental.pallas import tpu as pltpu

B, LQ, LKV = 2, 2048, 512
Q_DIM, KV_DIM, D_ATTN, H = 1024, 1024, 1024, 16
DH = D_ATTN // H
HALF = DH // 2
LOOKBACK = 7
SMAX = 8192
SCALE = DH ** -0.5

BQ = 512
NQ = LQ // BQ

_F32 = jnp.float32


def _rope_cs(pos_col):
    i = jax.lax.broadcasted_iota(jnp.int32, (1, HALF), 1).astype(_F32)
    inv_freq = 1.0 / jnp.power(10000.0, i * (2.0 / DH))
    freqs = pos_col * inv_freq
    cos = jnp.cos(freqs).astype(jnp.bfloat16).astype(_F32)
    sin = jnp.sin(freqs).astype(jnp.bfloat16).astype(_F32)
    return cos, sin


def _rope_head(x, cos, sin):
    x1 = x[:, :HALF]
    x2 = x[:, HALF:]
    return jnp.concatenate([x1 * cos - x2 * sin, x1 * sin + x2 * cos], axis=1)


def _attn_kernel(q_ref, kv_src_ref, seg_ref, qpos_ref, kvpos_ref,
                 wq_ref, wkv_ref, wo_ref, out_ref, kr_s, v_s):
    iq = pl.program_id(1)

    @pl.when(iq == 0)
    def _():
        kv = jax.lax.dot_general(
            kv_src_ref[0], wkv_ref[...],
            (((1,), (0,)), ((), ())), preferred_element_type=_F32)
        kpos = jnp.clip(kvpos_ref[...], 0.0, SMAX - 1.0)
        kcos, ksin = _rope_cs(kpos)
        for h in range(H):
            kh = kv[:, h * DH:(h + 1) * DH]
            kr_s[:, h * DH:(h + 1) * DH] = _rope_head(kh, kcos, ksin)
        v_s[...] = kv[:, D_ATTN:]

    qh = jax.lax.dot_general(
        q_ref[0], wq_ref[...],
        (((1,), (0,)), ((), ())), preferred_element_type=_F32)
    qpos = jnp.clip(qpos_ref[0], 0.0, SMAX - 1.0)
    qcos, qsin = _rope_cs(qpos)

    seg = seg_ref[0]
    jj = jax.lax.broadcasted_iota(jnp.int32, (LKV, BQ), 0).astype(_F32)
    mask = jnp.logical_and(jj <= seg, jj >= seg - float(LOOKBACK))
    neg_inf = float(jnp.finfo(_F32).min)

    outs = []
    for h in range(H):
        q_h = _rope_head(qh[:, h * DH:(h + 1) * DH], qcos, qsin) * SCALE
        k_h = kr_s[:, h * DH:(h + 1) * DH]
        s = jax.lax.dot_general(
            k_h, q_h, (((1,), (1,)), ((), ())),
            preferred_element_type=_F32)
        s = jnp.where(mask, s, neg_inf)
        m = jnp.max(s, axis=0, keepdims=True)
        p = jnp.exp(s - m)
        d = jnp.sum(p, axis=0, keepdims=True)
        p = p * (1.0 / d)
        o_h = jax.lax.dot_general(
            p, v_s[:, h * DH:(h + 1) * DH], (((0,), (0,)), ((), ())),
            preferred_element_type=_F32)
        outs.append(o_h)

    attn = jnp.concatenate(outs, axis=1)
    out_ref[0] = jax.lax.dot_general(
        attn, wo_ref[...], (((1,), (0,)), ((), ())),
        preferred_element_type=_F32)


@jax.jit
def kernel(q, kv_src, seg_id, q_pos_ids, kv_pos_ids, Wq, Wkv, Wo):
    seg_f = seg_id.astype(_F32).reshape(B, 1, LQ)
    qpos_f = q_pos_ids.astype(_F32).reshape(B, LQ, 1)
    kvpos_f = kv_pos_ids.astype(_F32).reshape(LKV, 1)

    grid = (B, NQ)
    out = pl.pallas_call(
        _attn_kernel,
        grid=grid,
        in_specs=[
            pl.BlockSpec((1, BQ, Q_DIM), lambda b, i: (b, i, 0)),
            pl.BlockSpec((1, LKV, KV_DIM), lambda b, i: (b, 0, 0)),
            pl.BlockSpec((1, 1, BQ), lambda b, i: (b, 0, i)),
            pl.BlockSpec((1, BQ, 1), lambda b, i: (b, i, 0)),
            pl.BlockSpec((LKV, 1), lambda b, i: (0, 0)),
            pl.BlockSpec((Q_DIM, D_ATTN), lambda b, i: (0, 0)),
            pl.BlockSpec((KV_DIM, 2 * D_ATTN), lambda b, i: (0, 0)),
            pl.BlockSpec((D_ATTN, Q_DIM), lambda b, i: (0, 0)),
        ],
        out_specs=pl.BlockSpec((1, BQ, Q_DIM), lambda b, i: (b, i, 0)),
        out_shape=jax.ShapeDtypeStruct((B, LQ, Q_DIM), _F32),
        scratch_shapes=[
            pltpu.VMEM((LKV, D_ATTN), _F32),
            pltpu.VMEM((LKV, D_ATTN), _F32),
        ],
    )(q, kv_src, seg_f, qpos_f, kvpos_f, Wq, Wkv, Wo)
    return out
